# Initial kernel scaffold; baseline (speedup 1.0000x reference)
import functools

import jax
import jax.numpy as jnp
from jax import lax
from jax.experimental import pallas as pl
from jax.experimental.pallas import tpu as pltpu
from jax.experimental.pallas import tpu_sc as plsc

NN = 10000
EE = 320000
DD = 128
HH = 64
CC = 3
GG = 128

NC = 2
NS = 16
NT = NC * NS
K = 128
CHUNKS = -(-EE // (NT * K))
EP = NT * K * CHUNKS
NP16 = 10240
NROW = NN // NS
DSLC = NP16 // NS

_f32 = jnp.float32
_i32 = jnp.int32


def _mesh():
    return plsc.VectorSubcoreMesh(core_axis_name="c", subcore_axis_name="s")


def _sc_deg(d2d, w2d):
    @functools.partial(
        pl.kernel,
        out_type=jax.ShapeDtypeStruct((NC, NP16), _f32),
        mesh=_mesh(),
        scratch_types=[
            pltpu.VMEM((CHUNKS, K), _i32),
            pltpu.VMEM((CHUNKS, K), _f32),
            pltpu.VMEM((DSLC,), _f32),
            pltpu.VMEM_SHARED((NP16,), _f32),
            pltpu.SemaphoreType.DMA,
        ],
    )
    def deg_kernel(d_hbm, w_hbm, out_hbm, dv, wv, zb, wsum_sp, sem):
        c = lax.axis_index("c")
        s = lax.axis_index("s")
        wid = c * NS + s
        pltpu.async_copy(d_hbm.at[pl.ds(wid * CHUNKS, CHUNKS)], dv, sem).wait()
        pltpu.async_copy(w_hbm.at[pl.ds(wid * CHUNKS, CHUNKS)], wv, sem).wait()
        zeros = jnp.zeros((16,), _f32)

        @pl.loop(0, DSLC, step=16)
        def _(i):
            zb[pl.ds(i, 16)] = zeros

        pltpu.sync_copy(zb, wsum_sp.at[pl.ds(s * DSLC, DSLC)])
        plsc.subcore_barrier()

        @pl.loop(0, CHUNKS)
        def _(k):
            pltpu.sync_copy(wv.at[k], wsum_sp.at[dv.at[k]], add=True)

        plsc.subcore_barrier()
        pltpu.sync_copy(wsum_sp.at[pl.ds(s * DSLC, DSLC)], zb)
        pltpu.sync_copy(zb, out_hbm.at[c, pl.ds(s * DSLC, DSLC)])

    return deg_kernel(d2d, w2d)


def _sc_spmm(table, s2d, d2d, w2d):
    @functools.partial(
        pl.kernel,
        out_type=jax.ShapeDtypeStruct((NC, NN, HH), _f32),
        mesh=_mesh(),
        scratch_types=[
            pltpu.VMEM((CHUNKS, K), _i32),
            pltpu.VMEM((CHUNKS, K), _i32),
            pltpu.VMEM((CHUNKS, K), _f32),
            pltpu.VMEM((K, HH), _f32),
            pltpu.VMEM((NROW, HH), _f32),
            pltpu.VMEM_SHARED((NN, HH), _f32),
            pltpu.VMEM_SHARED((NN, HH), _f32),
            pltpu.SemaphoreType.DMA,
            pltpu.SemaphoreType.DMA,
        ],
    )
    def spmm_kernel(t_hbm, s_hbm, d_hbm, w_hbm, out_hbm,
                    sv, dv, wv, rows, stage, table_sp, acc_sp, sem, sem2):
        c = lax.axis_index("c")
        s = lax.axis_index("s")
        wid = c * NS + s
        row0 = s * NROW

        pltpu.async_copy(s_hbm.at[pl.ds(wid * CHUNKS, CHUNKS)], sv, sem).wait()
        pltpu.async_copy(d_hbm.at[pl.ds(wid * CHUNKS, CHUNKS)], dv, sem).wait()
        pltpu.async_copy(w_hbm.at[pl.ds(wid * CHUNKS, CHUNKS)], wv, sem).wait()
        zeros = jnp.zeros((16,), _f32)

        @pl.loop(0, NROW)
        def _(r):
            for j in range(HH // 16):
                stage[r, pl.ds(j * 16, 16)] = zeros

        pltpu.sync_copy(stage, acc_sp.at[pl.ds(row0, NROW)])
        pltpu.async_copy(t_hbm.at[pl.ds(row0, NROW)], stage, sem).wait()
        pltpu.sync_copy(stage, table_sp.at[pl.ds(row0, NROW)])
        plsc.subcore_barrier()

        @pl.loop(0, CHUNKS)
        def _(k):
            pltpu.async_copy(table_sp.at[sv.at[k]], rows, sem2).wait()

            @pl.loop(0, K)
            def _(e):
                w = wv[k, e]
                for j in range(HH // 16):
                    sl = pl.ds(j * 16, 16)
                    rows[e, sl] = rows[e, sl] * w

            pltpu.sync_copy(rows, acc_sp.at[dv.at[k]], add=True)

        plsc.subcore_barrier()
        pltpu.sync_copy(acc_sp.at[pl.ds(row0, NROW)], stage)
        pltpu.sync_copy(stage, out_hbm.at[c, pl.ds(row0, NROW)])

    return spmm_kernel(table, s2d, d2d, w2d)


def _tc_matmul1(x, W1):
    def body(x_ref, w_ref, o_ref):
        o_ref[...] = jnp.dot(x_ref[...], w_ref[...],
                             preferred_element_type=_f32)

    return pl.pallas_call(
        body, out_shape=jax.ShapeDtypeStruct((NN, HH), _f32))(x, W1)


def _tc_scale(h0, ws):
    def body(h_ref, ws_ref, hs_ref, dinv_ref):
        deg = ws_ref[0] + ws_ref[1] + 1.0
        dinv = lax.rsqrt(deg)
        dinv_ref[...] = dinv
        hs_ref[...] = h_ref[...] * dinv

    return pl.pallas_call(
        body,
        out_shape=(jax.ShapeDtypeStruct((NN, HH), _f32),
                   jax.ShapeDtypeStruct((NN, 1), _f32)))(h0, ws)


def _tc_mid(p, h0s, dinv, b1, W2):
    def body(p_ref, hs_ref, di_ref, b_ref, w_ref, o_ref):
        dinv = di_ref[...]
        h1 = jnp.maximum((p_ref[0] + p_ref[1] + hs_ref[...]) * dinv
                         + b_ref[...], 0.0)
        o_ref[...] = jnp.dot(h1, w_ref[...],
                             preferred_element_type=_f32) * dinv

    return pl.pallas_call(
        body, out_shape=jax.ShapeDtypeStruct((NN, HH), _f32))(
            p, h0s, dinv, b1, W2)


def _tc_final(q, h1s, dinv, b2, batch_row, Wf1, bf1, Wf2, bf2):
    def body(q_ref, hs_ref, di_ref, b2_ref, bt_ref,
             wf1_ref, bf1_ref, wf2_ref, bf2_ref, o_ref):
        h2 = jnp.maximum((q_ref[0] + q_ref[1] + hs_ref[...]) * di_ref[...]
                         + b2_ref[...], 0.0)
        gids = lax.broadcasted_iota(_i32, (GG, NN), 0)
        onehot = (gids == bt_ref[...]).astype(_f32)
        sums = jnp.dot(onehot, h2, preferred_element_type=_f32)
        counts = jnp.sum(onehot, axis=1, keepdims=True)
        pooled = sums / jnp.maximum(counts, 1.0)
        hm = jnp.maximum(
            jnp.dot(pooled, wf1_ref[...], preferred_element_type=_f32)
            + bf1_ref[...], 0.0)
        o_ref[...] = (jnp.dot(hm, wf2_ref[...], preferred_element_type=_f32)
                      + bf2_ref[...])

    return pl.pallas_call(
        body, out_shape=jax.ShapeDtypeStruct((GG, CC), _f32))(
            q, h1s, dinv, b2, batch_row, Wf1, bf1, Wf2, bf2)


def kernel(x, edge_index, edge_weight, batch, W1, b1, W2, b2,
           Wf1, bf1, Wf2, bf2):
    src = edge_index[0]
    dst = edge_index[1]
    pad = EP - EE
    s2d = jnp.pad(src, (0, pad)).reshape(NT * CHUNKS, K)
    d2d = jnp.pad(dst, (0, pad)).reshape(NT * CHUNKS, K)
    w2d = jnp.pad(edge_weight, (0, pad)).reshape(NT * CHUNKS, K)

    ws = _sc_deg(d2d, w2d)
    h0 = _tc_matmul1(x, W1)
    h0s, dinv = _tc_scale(h0, ws[:, :NN, None])
    p = _sc_spmm(h0s, s2d, d2d, w2d)
    h1s = _tc_mid(p, h0s, dinv, b1.reshape(1, HH), W2)
    q = _sc_spmm(h1s, s2d, d2d, w2d)
    out = _tc_final(q, h1s, dinv, b2.reshape(1, HH), batch.reshape(1, NN),
                    Wf1, bf1.reshape(1, HH), Wf2, bf2.reshape(1, CC))
    return out


# speedup vs baseline: 9.4434x; 9.4434x over previous
import functools

import jax
import jax.numpy as jnp
from jax import lax
from jax.experimental import pallas as pl
from jax.experimental.pallas import tpu as pltpu
from jax.experimental.pallas import tpu_sc as plsc

NN = 10000
EE = 320000
DD = 128
HH = 64
CC = 3
GG = 128

NC = 2
NS = 16
NT = NC * NS
K = 128
CHUNKS = (-(-EE // (NT * K)) + 7) // 8 * 8
EP = NT * K * CHUNKS
NP = 10240
NROW = NP // NS
DSLC = NP // NS

_f32 = jnp.float32
_i32 = jnp.int32


def _mesh():
    return plsc.VectorSubcoreMesh(core_axis_name="c", subcore_axis_name="s")


def _sc_deg(d2d, w2d):
    @functools.partial(
        pl.kernel,
        out_type=jax.ShapeDtypeStruct((NC, NP), _f32),
        mesh=_mesh(),
        scratch_types=[
            pltpu.VMEM((CHUNKS, K), _i32),
            pltpu.VMEM((CHUNKS, K), _f32),
            pltpu.VMEM((DSLC,), _f32),
            pltpu.VMEM_SHARED((NP,), _f32),
            pltpu.SemaphoreType.DMA,
        ],
    )
    def deg_kernel(d_hbm, w_hbm, out_hbm, dv, wv, zb, wsum_sp, sem):
        c = lax.axis_index("c")
        s = lax.axis_index("s")
        wid = c * NS + s
        pltpu.async_copy(d_hbm.at[pl.ds(wid * CHUNKS, CHUNKS)], dv, sem).wait()
        pltpu.async_copy(w_hbm.at[pl.ds(wid * CHUNKS, CHUNKS)], wv, sem).wait()
        zeros = jnp.zeros((16,), _f32)

        @pl.loop(0, DSLC, step=16)
        def _(i):
            zb[pl.ds(i, 16)] = zeros

        pltpu.sync_copy(zb, wsum_sp.at[pl.ds(s * DSLC, DSLC)])
        plsc.subcore_barrier()

        @pl.loop(0, CHUNKS)
        def _(k):
            pltpu.sync_copy(wv.at[k], wsum_sp.at[dv.at[k]], add=True)

        plsc.subcore_barrier()
        pltpu.sync_copy(wsum_sp.at[pl.ds(s * DSLC, DSLC)], zb)
        pltpu.sync_copy(zb, out_hbm.at[c, pl.ds(s * DSLC, DSLC)])

    return deg_kernel(d2d, w2d)


def _sc_spmm(table, s2d, d2d, w2d):
    @functools.partial(
        pl.kernel,
        out_type=jax.ShapeDtypeStruct((NC, NP, HH), _f32),
        mesh=_mesh(),
        scratch_types=[
            pltpu.VMEM((8, K), _i32),
            pltpu.VMEM((8, K), _i32),
            pltpu.VMEM((8, K), _f32),
            pltpu.VMEM((K, HH), _f32),
            pltpu.VMEM((K, HH), _f32),
            pltpu.VMEM_SHARED((NP, HH), _f32),
            pltpu.SemaphoreType.DMA,
            pltpu.SemaphoreType.DMA,
        ],
        compiler_params=pltpu.CompilerParams(use_tc_tiling_on_sc=False),
    )
    def spmm_kernel(t_hbm, s_hbm, d_hbm, w_hbm, out_hbm,
                    sv, dv, wv, rows, stage, acc_sp, sem, sem2):
        c = lax.axis_index("c")
        s = lax.axis_index("s")
        wid = c * NS + s
        row0 = s * NROW

        zeros = jnp.zeros((16,), _f32)

        @pl.loop(0, K)
        def _(r):
            for j in range(HH // 16):
                stage[r, pl.ds(j * 16, 16)] = zeros

        for t in range(NROW // K):
            pltpu.sync_copy(stage, acc_sp.at[pl.ds(row0 + t * K, K)])
        plsc.subcore_barrier()

        @pl.loop(0, CHUNKS // 8)
        def _(g):
            base = pl.multiple_of(wid * CHUNKS + g * 8, 8)
            c1 = pltpu.async_copy(s_hbm.at[pl.ds(base, 8)], sv, sem)
            c2 = pltpu.async_copy(d_hbm.at[pl.ds(base, 8)], dv, sem)
            c3 = pltpu.async_copy(w_hbm.at[pl.ds(base, 8)], wv, sem)
            c1.wait()
            c2.wait()
            c3.wait()

            @pl.loop(0, 8)
            def _(k):
                pltpu.async_copy(t_hbm.at[sv.at[k]], rows, sem2).wait()

                @pl.loop(0, K, step=16)
                def _(e0):
                    wg = wv[k, pl.ds(e0, 16)]
                    for i in range(16):
                        w = wg[i]
                        for j in range(HH // 16):
                            sl = pl.ds(j * 16, 16)
                            rows[e0 + i, sl] = rows[e0 + i, sl] * w

                pltpu.sync_copy(rows, acc_sp.at[dv.at[k]], add=True)

        plsc.subcore_barrier()
        for t in range(NROW // K):
            pltpu.sync_copy(acc_sp.at[pl.ds(row0 + t * K, K)], stage)
            pltpu.sync_copy(stage, out_hbm.at[c, pl.ds(row0 + t * K, K)])

    return spmm_kernel(table, s2d, d2d, w2d)


def _tc_matmul1(x, W1):
    def body(x_ref, w_ref, o_ref):
        o_ref[...] = jnp.dot(x_ref[...], w_ref[...],
                             preferred_element_type=_f32)

    return pl.pallas_call(
        body, out_shape=jax.ShapeDtypeStruct((NP, HH), _f32))(x, W1)


def _tc_scale(h0, ws):
    def body(h_ref, ws_ref, hs_ref, dinv_ref):
        deg = ws_ref[0] + ws_ref[1] + 1.0
        dinv = lax.rsqrt(deg)
        dinv_ref[...] = dinv
        hs_ref[...] = h_ref[...] * dinv

    return pl.pallas_call(
        body,
        out_shape=(jax.ShapeDtypeStruct((NP, HH), _f32),
                   jax.ShapeDtypeStruct((NP, 1), _f32)))(h0, ws)


def _tc_mid(p, h0s, dinv, b1, W2):
    def body(p_ref, hs_ref, di_ref, b_ref, w_ref, o_ref):
        dinv = di_ref[...]
        h1 = jnp.maximum((p_ref[0] + p_ref[1] + hs_ref[...]) * dinv
                         + b_ref[...], 0.0)
        o_ref[...] = jnp.dot(h1, w_ref[...],
                             preferred_element_type=_f32) * dinv

    return pl.pallas_call(
        body, out_shape=jax.ShapeDtypeStruct((NP, HH), _f32))(
            p, h0s, dinv, b1, W2)


def _tc_final(q, h1s, dinv, b2, batch_row, Wf1, bf1, Wf2, bf2):
    def body(q_ref, hs_ref, di_ref, b2_ref, bt_ref,
             wf1_ref, bf1_ref, wf2_ref, bf2_ref, o_ref):
        h2 = jnp.maximum((q_ref[0] + q_ref[1] + hs_ref[...]) * di_ref[...]
                         + b2_ref[...], 0.0)
        gids = lax.broadcasted_iota(_i32, (GG, NP), 0)
        onehot = (gids == bt_ref[...]).astype(_f32)
        sums = jnp.dot(onehot, h2, preferred_element_type=_f32)
        counts = jnp.sum(onehot, axis=1, keepdims=True)
        pooled = sums / jnp.maximum(counts, 1.0)
        hm = jnp.maximum(
            jnp.dot(pooled, wf1_ref[...], preferred_element_type=_f32)
            + bf1_ref[...], 0.0)
        o_ref[...] = (jnp.dot(hm, wf2_ref[...], preferred_element_type=_f32)
                      + bf2_ref[...])

    return pl.pallas_call(
        body, out_shape=jax.ShapeDtypeStruct((GG, CC), _f32))(
            q, h1s, dinv, b2, batch_row, Wf1, bf1, Wf2, bf2)


def kernel(x, edge_index, edge_weight, batch, W1, b1, W2, b2,
           Wf1, bf1, Wf2, bf2):
    src = edge_index[0]
    dst = edge_index[1]
    pad = EP - EE
    s2d = jnp.pad(src, (0, pad)).reshape(NT * CHUNKS, K)
    d2d = jnp.pad(dst, (0, pad)).reshape(NT * CHUNKS, K)
    w2d = jnp.pad(edge_weight, (0, pad)).reshape(NT * CHUNKS, K)
    x_p = jnp.pad(x, ((0, NP - NN), (0, 0)))
    batch_p = jnp.pad(batch, (0, NP - NN),
                      constant_values=GG).reshape(1, NP)

    ws = _sc_deg(d2d, w2d)
    h0 = _tc_matmul1(x_p, W1)
    h0s, dinv = _tc_scale(h0, ws[:, :, None])
    p = _sc_spmm(h0s, s2d, d2d, w2d)
    h1s = _tc_mid(p, h0s, dinv, b1.reshape(1, HH), W2)
    q = _sc_spmm(h1s, s2d, d2d, w2d)
    out = _tc_final(q, h1s, dinv, b2.reshape(1, HH), batch_p,
                    Wf1, bf1.reshape(1, HH), Wf2, bf2.reshape(1, CC))
    return out


# speedup vs baseline: 14.4755x; 1.5329x over previous
import functools

import jax
import jax.numpy as jnp
from jax import lax
from jax.experimental import pallas as pl
from jax.experimental.pallas import tpu as pltpu
from jax.experimental.pallas import tpu_sc as plsc

NN = 10000
EE = 320000
DD = 128
HH = 64
CC = 3
GG = 128

NC = 2
NS = 16
NT = NC * NS
K = 128
CHUNKS = (-(-EE // (NT * K)) + 7) // 8 * 8
EP = NT * K * CHUNKS
NP = 10240
NROW = NP // NS
DSLC = NP // NS

_f32 = jnp.float32
_i32 = jnp.int32


def _mesh():
    return plsc.VectorSubcoreMesh(core_axis_name="c", subcore_axis_name="s")


def _sc_deg(d2d, w2d):
    @functools.partial(
        pl.kernel,
        out_type=jax.ShapeDtypeStruct((NC, NP), _f32),
        mesh=_mesh(),
        scratch_types=[
            pltpu.VMEM((CHUNKS, K), _i32),
            pltpu.VMEM((CHUNKS, K), _f32),
            pltpu.VMEM((DSLC,), _f32),
            pltpu.VMEM_SHARED((NP,), _f32),
            pltpu.SemaphoreType.DMA,
        ],
    )
    def deg_kernel(d_hbm, w_hbm, out_hbm, dv, wv, zb, wsum_sp, sem):
        c = lax.axis_index("c")
        s = lax.axis_index("s")
        wid = c * NS + s
        pltpu.async_copy(d_hbm.at[pl.ds(wid * CHUNKS, CHUNKS)], dv, sem).wait()
        pltpu.async_copy(w_hbm.at[pl.ds(wid * CHUNKS, CHUNKS)], wv, sem).wait()
        zeros = jnp.zeros((16,), _f32)

        @pl.loop(0, DSLC, step=16)
        def _(i):
            zb[pl.ds(i, 16)] = zeros

        pltpu.sync_copy(zb, wsum_sp.at[pl.ds(s * DSLC, DSLC)])
        plsc.subcore_barrier()

        @pl.loop(0, CHUNKS)
        def _(k):
            pltpu.sync_copy(wv.at[k], wsum_sp.at[dv.at[k]], add=True)

        plsc.subcore_barrier()
        pltpu.sync_copy(wsum_sp.at[pl.ds(s * DSLC, DSLC)], zb)
        pltpu.sync_copy(zb, out_hbm.at[c, pl.ds(s * DSLC, DSLC)])

    return deg_kernel(d2d, w2d)


def _sc_spmm(table, s2d, d2d, w2d):
    @functools.partial(
        pl.kernel,
        out_type=jax.ShapeDtypeStruct((NC, NP, HH), _f32),
        mesh=_mesh(),
        scratch_types=[
            pltpu.VMEM((CHUNKS, K), _i32),
            pltpu.VMEM((CHUNKS, K), _i32),
            pltpu.VMEM((CHUNKS, K), _f32),
            pltpu.VMEM((3, K, HH), _f32),
            pltpu.VMEM((K, HH), _f32),
            pltpu.VMEM_SHARED((NP, HH), _f32),
            pltpu.SemaphoreType.DMA,
            pltpu.SemaphoreType.DMA,
        ],
        compiler_params=pltpu.CompilerParams(use_tc_tiling_on_sc=False),
    )
    def spmm_kernel(t_hbm, s_hbm, d_hbm, w_hbm, out_hbm,
                    sv, dv, wv, rows, stage, acc_sp, sem, sem2):
        c = lax.axis_index("c")
        s = lax.axis_index("s")
        wid = c * NS + s
        row0 = s * NROW

        ce1 = pltpu.async_copy(s_hbm.at[pl.ds(wid * CHUNKS, CHUNKS)], sv, sem)
        ce2 = pltpu.async_copy(d_hbm.at[pl.ds(wid * CHUNKS, CHUNKS)], dv, sem)
        ce3 = pltpu.async_copy(w_hbm.at[pl.ds(wid * CHUNKS, CHUNKS)], wv, sem)
        zeros = jnp.zeros((16,), _f32)

        @pl.loop(0, K)
        def _(r):
            for j in range(HH // 16):
                stage[r, pl.ds(j * 16, 16)] = zeros

        for t in range(NROW // K):
            pltpu.sync_copy(stage, acc_sp.at[pl.ds(row0 + t * K, K)])
        ce1.wait()
        ce2.wait()
        ce3.wait()
        plsc.subcore_barrier()

        pltpu.async_copy(t_hbm.at[sv.at[0]], rows.at[0], sem2)
        pltpu.async_copy(t_hbm.at[sv.at[1]], rows.at[1], sem2)

        @pl.loop(0, CHUNKS)
        def _(k):
            b = lax.rem(k, 3)
            pltpu.make_async_copy(t_hbm.at[sv.at[k]], rows.at[b], sem2).wait()

            @pl.when(k < CHUNKS - 2)
            def _():
                pltpu.async_copy(t_hbm.at[sv.at[k + 2]],
                                 rows.at[lax.rem(k + 2, 3)], sem2)

            rb = rows.at[b]

            @pl.loop(0, K, step=16)
            def _(e0):
                wg = wv[k, pl.ds(e0, 16)]
                for i in range(16):
                    w = wg[i]
                    for j in range(HH // 16):
                        sl = pl.ds(j * 16, 16)
                        rb[e0 + i, sl] = rb[e0 + i, sl] * w

            pltpu.sync_copy(rb, acc_sp.at[dv.at[k]], add=True)

        plsc.subcore_barrier()
        for t in range(NROW // K):
            pltpu.sync_copy(acc_sp.at[pl.ds(row0 + t * K, K)], stage)
            pltpu.sync_copy(stage, out_hbm.at[c, pl.ds(row0 + t * K, K)])

    return spmm_kernel(table, s2d, d2d, w2d)


def _tc_matmul1(x, W1):
    def body(x_ref, w_ref, o_ref):
        o_ref[...] = jnp.dot(x_ref[...], w_ref[...],
                             preferred_element_type=_f32)

    return pl.pallas_call(
        body, out_shape=jax.ShapeDtypeStruct((NP, HH), _f32))(x, W1)


def _tc_scale(h0, ws):
    def body(h_ref, ws_ref, hs_ref, dinv_ref):
        deg = ws_ref[0] + ws_ref[1] + 1.0
        dinv = 1.0 / jnp.sqrt(deg)
        dinv_ref[...] = dinv
        hs_ref[...] = h_ref[...] * dinv

    return pl.pallas_call(
        body,
        out_shape=(jax.ShapeDtypeStruct((NP, HH), _f32),
                   jax.ShapeDtypeStruct((NP, 1), _f32)))(h0, ws)


def _tc_mid(p, h0s, dinv, b1, W2):
    def body(p_ref, hs_ref, di_ref, b_ref, w_ref, o_ref):
        dinv = di_ref[...]
        h1 = jnp.maximum((p_ref[0] + p_ref[1] + hs_ref[...]) * dinv
                         + b_ref[...], 0.0)
        o_ref[...] = jnp.dot(h1, w_ref[...],
                             preferred_element_type=_f32) * dinv

    return pl.pallas_call(
        body, out_shape=jax.ShapeDtypeStruct((NP, HH), _f32))(
            p, h0s, dinv, b1, W2)


def _tc_final(q, h1s, dinv, b2, batch_row, Wf1, bf1, Wf2, bf2):
    def body(q_ref, hs_ref, di_ref, b2_ref, bt_ref,
             wf1_ref, bf1_ref, wf2_ref, bf2_ref, o_ref):
        h2 = jnp.maximum((q_ref[0] + q_ref[1] + hs_ref[...]) * di_ref[...]
                         + b2_ref[...], 0.0)
        gids = lax.broadcasted_iota(_i32, (GG, NP), 0)
        onehot = (gids == bt_ref[...]).astype(_f32)
        sums = jnp.dot(onehot, h2, preferred_element_type=_f32,
                       precision=lax.Precision.HIGHEST)
        counts = jnp.sum(onehot, axis=1, keepdims=True)
        pooled = sums / jnp.maximum(counts, 1.0)
        hm = jnp.maximum(
            jnp.dot(pooled, wf1_ref[...], preferred_element_type=_f32)
            + bf1_ref[...], 0.0)
        o_ref[...] = (jnp.dot(hm, wf2_ref[...], preferred_element_type=_f32)
                      + bf2_ref[...])

    return pl.pallas_call(
        body, out_shape=jax.ShapeDtypeStruct((GG, CC), _f32))(
            q, h1s, dinv, b2, batch_row, Wf1, bf1, Wf2, bf2)


def kernel(x, edge_index, edge_weight, batch, W1, b1, W2, b2,
           Wf1, bf1, Wf2, bf2):
    src = edge_index[0]
    dst = edge_index[1]
    pad = EP - EE
    s2d = jnp.pad(src, (0, pad)).reshape(NT * CHUNKS, K)
    d2d = jnp.pad(dst, (0, pad)).reshape(NT * CHUNKS, K)
    w2d = jnp.pad(edge_weight, (0, pad)).reshape(NT * CHUNKS, K)
    x_p = jnp.pad(x, ((0, NP - NN), (0, 0)))
    batch_p = jnp.pad(batch, (0, NP - NN),
                      constant_values=GG).reshape(1, NP)

    ws = _sc_deg(d2d, w2d)
    h0 = _tc_matmul1(x_p, W1)
    h0s, dinv = _tc_scale(h0, ws[:, :, None])
    p = _sc_spmm(h0s, s2d, d2d, w2d)
    h1s = _tc_mid(p, h0s, dinv, b1.reshape(1, HH), W2)
    q = _sc_spmm(h1s, s2d, d2d, w2d)
    out = _tc_final(q, h1s, dinv, b2.reshape(1, HH), batch_p,
                    Wf1, bf1.reshape(1, HH), Wf2, bf2.reshape(1, CC))
    return out


# speedup vs baseline: 14.6349x; 1.0110x over previous
import functools

import jax
import jax.numpy as jnp
from jax import lax
from jax.experimental import pallas as pl
from jax.experimental.pallas import tpu as pltpu
from jax.experimental.pallas import tpu_sc as plsc

NN = 10000
EE = 320000
DD = 128
HH = 64
CC = 3
GG = 128

NC = 2
NS = 16
NT = NC * NS
K = 128
CHUNKS = (-(-EE // (NT * K)) + 7) // 8 * 8
EP = NT * K * CHUNKS
NP = 10240
NROW = NP // NS
DSLC = NP // NS

_f32 = jnp.float32
_i32 = jnp.int32


def _mesh():
    return plsc.VectorSubcoreMesh(core_axis_name="c", subcore_axis_name="s")


def _sc_deg(d2d, w2d):
    @functools.partial(
        pl.kernel,
        out_type=jax.ShapeDtypeStruct((NC, NP), _f32),
        mesh=_mesh(),
        scratch_types=[
            pltpu.VMEM((CHUNKS, K), _i32),
            pltpu.VMEM((CHUNKS, K), _f32),
            pltpu.VMEM((DSLC,), _f32),
            pltpu.VMEM_SHARED((NP,), _f32),
            pltpu.SemaphoreType.DMA,
        ],
    )
    def deg_kernel(d_hbm, w_hbm, out_hbm, dv, wv, zb, wsum_sp, sem):
        c = lax.axis_index("c")
        s = lax.axis_index("s")
        wid = c * NS + s
        pltpu.async_copy(d_hbm.at[pl.ds(wid * CHUNKS, CHUNKS)], dv, sem).wait()
        pltpu.async_copy(w_hbm.at[pl.ds(wid * CHUNKS, CHUNKS)], wv, sem).wait()
        zeros = jnp.zeros((16,), _f32)

        @pl.loop(0, DSLC, step=16)
        def _(i):
            zb[pl.ds(i, 16)] = zeros

        pltpu.sync_copy(zb, wsum_sp.at[pl.ds(s * DSLC, DSLC)])
        plsc.subcore_barrier()

        @pl.loop(0, CHUNKS)
        def _(k):
            pltpu.sync_copy(wv.at[k], wsum_sp.at[dv.at[k]], add=True)

        plsc.subcore_barrier()
        pltpu.sync_copy(wsum_sp.at[pl.ds(s * DSLC, DSLC)], zb)
        pltpu.sync_copy(zb, out_hbm.at[c, pl.ds(s * DSLC, DSLC)])

    return deg_kernel(d2d, w2d)


def _sc_spmm(table, s2d, d2d, w2d):
    @functools.partial(
        pl.kernel,
        out_type=jax.ShapeDtypeStruct((NC, NP, HH), _f32),
        mesh=_mesh(),
        scratch_types=[
            pltpu.VMEM((CHUNKS, K), _i32),
            pltpu.VMEM((CHUNKS, K), _i32),
            pltpu.VMEM((CHUNKS, K), _f32),
            pltpu.VMEM((3, K, HH), _f32),
            pltpu.VMEM((K, HH), _f32),
            pltpu.VMEM_SHARED((NP, HH), _f32),
            pltpu.SemaphoreType.DMA,
            pltpu.SemaphoreType.DMA,
            pltpu.SemaphoreType.DMA,
        ],
        compiler_params=pltpu.CompilerParams(use_tc_tiling_on_sc=False),
    )
    def spmm_kernel(t_hbm, s_hbm, d_hbm, w_hbm, out_hbm,
                    sv, dv, wv, rows, stage, acc_sp, sem, sem2, sem3):
        c = lax.axis_index("c")
        s = lax.axis_index("s")
        wid = c * NS + s
        row0 = s * NROW

        ce1 = pltpu.async_copy(s_hbm.at[pl.ds(wid * CHUNKS, CHUNKS)], sv, sem)
        ce2 = pltpu.async_copy(d_hbm.at[pl.ds(wid * CHUNKS, CHUNKS)], dv, sem)
        ce3 = pltpu.async_copy(w_hbm.at[pl.ds(wid * CHUNKS, CHUNKS)], wv, sem)
        zeros = jnp.zeros((16,), _f32)

        @pl.loop(0, K)
        def _(r):
            for j in range(HH // 16):
                stage[r, pl.ds(j * 16, 16)] = zeros

        for t in range(NROW // K):
            pltpu.sync_copy(stage, acc_sp.at[pl.ds(row0 + t * K, K)])
        ce1.wait()
        ce2.wait()
        ce3.wait()
        plsc.subcore_barrier()

        pltpu.async_copy(t_hbm.at[sv.at[0]], rows.at[0], sem2)
        pltpu.async_copy(t_hbm.at[sv.at[1]], rows.at[1], sem2)

        @pl.loop(0, CHUNKS)
        def _(k):
            b = lax.rem(k, 3)
            pltpu.make_async_copy(t_hbm.at[sv.at[k]], rows.at[b], sem2).wait()
            rb = rows.at[b]

            @pl.loop(0, K, step=16)
            def _(e0):
                wg = wv[k, pl.ds(e0, 16)]
                for i in range(16):
                    w = wg[i]
                    for j in range(HH // 16):
                        sl = pl.ds(j * 16, 16)
                        rb[e0 + i, sl] = rb[e0 + i, sl] * w

            @pl.when(k > 0)
            def _():
                pltpu.make_async_copy(
                    rows.at[lax.rem(k + 2, 3)],
                    acc_sp.at[dv.at[k - 1]], sem3).wait()

            @pl.when(k < CHUNKS - 2)
            def _():
                pltpu.async_copy(t_hbm.at[sv.at[k + 2]],
                                 rows.at[lax.rem(k + 2, 3)], sem2)

            pltpu.async_copy(rb, acc_sp.at[dv.at[k]], sem3, add=True)

        pltpu.make_async_copy(rows.at[lax.rem(CHUNKS - 1, 3)],
                              acc_sp.at[dv.at[CHUNKS - 1]], sem3).wait()
        plsc.subcore_barrier()
        for t in range(NROW // K):
            pltpu.sync_copy(acc_sp.at[pl.ds(row0 + t * K, K)], stage)
            pltpu.sync_copy(stage, out_hbm.at[c, pl.ds(row0 + t * K, K)])

    return spmm_kernel(table, s2d, d2d, w2d)


def _tc_matmul1(x, W1):
    def body(x_ref, w_ref, o_ref):
        o_ref[...] = jnp.dot(x_ref[...], w_ref[...],
                             preferred_element_type=_f32)

    return pl.pallas_call(
        body, out_shape=jax.ShapeDtypeStruct((NP, HH), _f32))(x, W1)


def _tc_scale(h0, ws):
    def body(h_ref, ws_ref, hs_ref, dinv_ref):
        deg = ws_ref[0] + ws_ref[1] + 1.0
        dinv = 1.0 / jnp.sqrt(deg)
        dinv_ref[...] = dinv
        hs_ref[...] = h_ref[...] * dinv

    return pl.pallas_call(
        body,
        out_shape=(jax.ShapeDtypeStruct((NP, HH), _f32),
                   jax.ShapeDtypeStruct((NP, 1), _f32)))(h0, ws)


def _tc_mid(p, h0s, dinv, b1, W2):
    def body(p_ref, hs_ref, di_ref, b_ref, w_ref, o_ref):
        dinv = di_ref[...]
        h1 = jnp.maximum((p_ref[0] + p_ref[1] + hs_ref[...]) * dinv
                         + b_ref[...], 0.0)
        o_ref[...] = jnp.dot(h1, w_ref[...],
                             preferred_element_type=_f32) * dinv

    return pl.pallas_call(
        body, out_shape=jax.ShapeDtypeStruct((NP, HH), _f32))(
            p, h0s, dinv, b1, W2)


def _tc_final(q, h1s, dinv, b2, batch_row, Wf1, bf1, Wf2, bf2):
    def body(q_ref, hs_ref, di_ref, b2_ref, bt_ref,
             wf1_ref, bf1_ref, wf2_ref, bf2_ref, o_ref):
        h2 = jnp.maximum((q_ref[0] + q_ref[1] + hs_ref[...]) * di_ref[...]
                         + b2_ref[...], 0.0)
        gids = lax.broadcasted_iota(_i32, (GG, NP), 0)
        onehot = (gids == bt_ref[...]).astype(_f32)
        sums = jnp.dot(onehot, h2, preferred_element_type=_f32,
                       precision=lax.Precision.HIGHEST)
        counts = jnp.sum(onehot, axis=1, keepdims=True)
        pooled = sums / jnp.maximum(counts, 1.0)
        hm = jnp.maximum(
            jnp.dot(pooled, wf1_ref[...], preferred_element_type=_f32)
            + bf1_ref[...], 0.0)
        o_ref[...] = (jnp.dot(hm, wf2_ref[...], preferred_element_type=_f32)
                      + bf2_ref[...])

    return pl.pallas_call(
        body, out_shape=jax.ShapeDtypeStruct((GG, CC), _f32))(
            q, h1s, dinv, b2, batch_row, Wf1, bf1, Wf2, bf2)


def kernel(x, edge_index, edge_weight, batch, W1, b1, W2, b2,
           Wf1, bf1, Wf2, bf2):
    src = edge_index[0]
    dst = edge_index[1]
    pad = EP - EE
    s2d = jnp.pad(src, (0, pad)).reshape(NT * CHUNKS, K)
    d2d = jnp.pad(dst, (0, pad)).reshape(NT * CHUNKS, K)
    w2d = jnp.pad(edge_weight, (0, pad)).reshape(NT * CHUNKS, K)
    x_p = jnp.pad(x, ((0, NP - NN), (0, 0)))
    batch_p = jnp.pad(batch, (0, NP - NN),
                      constant_values=GG).reshape(1, NP)

    ws = _sc_deg(d2d, w2d)
    h0 = _tc_matmul1(x_p, W1)
    h0s, dinv = _tc_scale(h0, ws[:, :, None])
    p = _sc_spmm(h0s, s2d, d2d, w2d)
    h1s = _tc_mid(p, h0s, dinv, b1.reshape(1, HH), W2)
    q = _sc_spmm(h1s, s2d, d2d, w2d)
    out = _tc_final(q, h1s, dinv, b2.reshape(1, HH), batch_p,
                    Wf1, bf1.reshape(1, HH), Wf2, bf2.reshape(1, CC))
    return out
